# baseline (speedup 1.0000x reference)
import functools

import jax
import jax.numpy as jnp
from jax import lax
from jax.experimental import pallas as pl
from jax.experimental.pallas import tpu as pltpu
from jax.experimental.pallas import tpu_sc as plsc

N_NODES = 10000
N_EDGES = 320000
D_FEAT = 128
D_EDGE = 16
UNITS = 128
W3 = 384

NCORE = 2
NSUB = 16
NWORK = NCORE * NSUB
BLK = 80
NBLK_A = 62
NBLK_B = 63
HALF_A = NWORK * BLK * NBLK_A
HALF_B = NWORK * BLK * NBLK_B
NPAD = 10240
STRIPE = NPAD // NSUB
CH = 32
NCH = STRIPE // CH
BE = 2560


def _sc_gather_counts(x, src, dst, estart, nblk):
    mesh = plsc.VectorSubcoreMesh(core_axis_name="c", subcore_axis_name="s")
    epw = BLK * nblk
    nedges = NWORK * epw

    @functools.partial(
        pl.kernel,
        mesh=mesh,
        out_type=[
            jax.ShapeDtypeStruct((nedges, D_FEAT), jnp.float32),
            jax.ShapeDtypeStruct((NCORE * NPAD, 128), jnp.float32),
        ],
        scratch_types=[
            pltpu.VMEM((3, BLK), jnp.int32),
            pltpu.VMEM((3, BLK), jnp.int32),
            pltpu.VMEM((BLK, D_FEAT), jnp.float32),
            pltpu.VMEM((BLK, D_FEAT), jnp.float32),
            pltpu.VMEM((BLK, D_FEAT), jnp.float32),
            pltpu.VMEM((BLK, 128), jnp.float32),
            pltpu.VMEM((CH, 128), jnp.float32),
            pltpu.VMEM_SHARED((NPAD, 128), jnp.float32),
            pltpu.SemaphoreType.DMA,
            pltpu.SemaphoreType.DMA,
            pltpu.SemaphoreType.DMA,
            pltpu.SemaphoreType.DMA,
        ],
    )
    def k(x_hbm, src_hbm, dst_hbm, g_hbm, cnt_hbm,
          src_r, dst_r, g_v0, g_v1, g_v2, ones_v, z_v, cnt_sh,
          sem_g, sem_w, sem_c, sem_i):
        cid = lax.axis_index("c")
        sid = lax.axis_index("s")
        ebase = estart + (cid * NSUB + sid) * epw
        obase = (cid * NSUB + sid) * epw
        gs = (g_v0, g_v1, g_v2)

        @pl.loop(0, CH)
        def _(r):
            @pl.loop(0, 128, step=16)
            def _(c0):
                z_v.at[r, pl.ds(c0, 16)][...] = jnp.zeros((16,), jnp.float32)

        @pl.loop(0, BLK)
        def _(r):
            @pl.loop(0, 128, step=16)
            def _(c0):
                ones_v.at[r, pl.ds(c0, 16)][...] = jnp.ones((16,), jnp.float32)

        row0 = sid * STRIPE
        for kk in range(NCH):
            pltpu.sync_copy(z_v, cnt_sh.at[pl.ds(row0 + kk * CH, CH), :])
        plsc.subcore_barrier()

        def fire_idx(b, r):
            pltpu.async_copy(src_hbm.at[pl.ds(ebase + b * BLK, BLK)],
                             src_r.at[r], sem_i)

        def wait_idx(r):
            pltpu.make_async_copy(src_hbm.at[pl.ds(ebase, BLK)],
                                  src_r.at[r], sem_i).wait()

        def fire_gather(b, r):
            pltpu.async_copy(x_hbm.at[src_r.at[r]], gs[r], sem_g)

        def wait_gather(r):
            pltpu.make_async_copy(x_hbm.at[src_r.at[r]], gs[r], sem_g).wait()

        def fire_write(b, r):
            pltpu.async_copy(gs[r], g_hbm.at[pl.ds(obase + b * BLK, BLK), :],
                             sem_w)

        def wait_write(b, r):
            pltpu.make_async_copy(
                gs[r], g_hbm.at[pl.ds(obase + b * BLK, BLK), :], sem_w).wait()

        def fire_cnt(b, r):
            pltpu.sync_copy(dst_hbm.at[pl.ds(ebase + b * BLK, BLK)],
                            dst_r.at[r])
            pltpu.async_copy(ones_v, cnt_sh.at[dst_r.at[r]], sem_c, add=True)

        def wait_cnt(r):
            pltpu.make_async_copy(ones_v, cnt_sh.at[dst_r.at[r]],
                                  sem_c).wait()

        def step(b, r, prefetch=True):
            r2 = (r + 2) % 3
            wait_gather(r)
            fire_write(b, r)
            if prefetch:
                fire_idx(b + 3, r)
            wait_write(b - 1, r2)
            wait_idx(r2)
            fire_gather(b + 2, r2)
            wait_cnt(r2)
            fire_cnt(b + 2, r2)

        fire_idx(0, 0); fire_idx(1, 1); fire_idx(2, 2)
        wait_idx(0); fire_gather(0, 0); fire_cnt(0, 0)
        wait_idx(1); fire_gather(1, 1); fire_cnt(1, 1)
        wait_gather(0); fire_write(0, 0); fire_idx(3, 0)
        wait_idx(2); fire_gather(2, 2); fire_cnt(2, 2)
        wait_gather(1); fire_write(1, 1); fire_idx(4, 1); wait_write(0, 0)
        wait_idx(0); fire_gather(3, 0); wait_cnt(0); fire_cnt(3, 0)
        wait_gather(2); fire_write(2, 2); fire_idx(5, 2); wait_write(1, 1)
        wait_idx(1); fire_gather(4, 1); wait_cnt(1); fire_cnt(4, 1)

        m = (nblk - 6) // 3

        @pl.loop(3, 3 + 3 * m, step=3)
        def _(g):
            for r in range(3):
                step(g + r, r)

        for b in range(3 + 3 * m, nblk - 3):
            step(b, b % 3)

        step(nblk - 3, (nblk - 3) % 3, prefetch=False)

        ra = (nblk - 2) % 3
        rb = (nblk - 1) % 3
        rc = (nblk - 3) % 3
        wait_gather(ra); fire_write(nblk - 2, ra); wait_write(nblk - 3, rc)
        wait_gather(rb); fire_write(nblk - 1, rb); wait_write(nblk - 2, ra)
        wait_write(nblk - 1, rb)
        wait_cnt(0); wait_cnt(1); wait_cnt(2)

        plsc.subcore_barrier()

        out0 = cid * NPAD + row0
        for kk in range(NCH):
            pltpu.sync_copy(cnt_sh.at[pl.ds(row0 + kk * CH, CH), :],
                            cnt_hbm.at[pl.ds(out0 + kk * CH, CH), :])

    return k(x, src, dst)


def _edge_stage(e_feats, g, we1, be1, we2, be2, kcat, wf, eoff):
    def body(e_ref, g_ref, w1_ref, b1_ref, w2_ref, b2_ref, kc_ref, wf_ref,
             o_ref):
        h = jnp.maximum(
            jnp.dot(e_ref[...], w1_ref[...], preferred_element_type=jnp.float32)
            + b1_ref[...], 0.0)
        w = jnp.dot(h, w2_ref[...], preferred_element_type=jnp.float32) + b2_ref[...]
        gk = jnp.dot(g_ref[...], kc_ref[...], preferred_element_type=jnp.float32)
        o_ref[...] = jnp.dot(w * gk, wf_ref[...],
                             preferred_element_type=jnp.float32)

    nedges = g.shape[0]
    off = eoff // BE
    return pl.pallas_call(
        body,
        grid=(nedges // BE,),
        in_specs=[
            pl.BlockSpec((BE, D_EDGE), lambda i: (i + off, 0)),
            pl.BlockSpec((BE, D_FEAT), lambda i: (i, 0)),
            pl.BlockSpec((D_EDGE, UNITS), lambda i: (0, 0)),
            pl.BlockSpec((1, UNITS), lambda i: (0, 0)),
            pl.BlockSpec((UNITS, W3), lambda i: (0, 0)),
            pl.BlockSpec((1, W3), lambda i: (0, 0)),
            pl.BlockSpec((D_FEAT, W3), lambda i: (0, 0)),
            pl.BlockSpec((W3, UNITS), lambda i: (0, 0)),
        ],
        out_specs=pl.BlockSpec((BE, UNITS), lambda i: (i, 0)),
        out_shape=jax.ShapeDtypeStruct((nedges, UNITS), jnp.float32),
    )(e_feats, g, we1, be1, we2, be2, kcat, wf)


def _sc_scatter(z, dst, estart, nblk):
    mesh = plsc.VectorSubcoreMesh(core_axis_name="c", subcore_axis_name="s")
    epw = BLK * nblk

    @functools.partial(
        pl.kernel,
        mesh=mesh,
        out_type=jax.ShapeDtypeStruct((NCORE * NPAD, 128), jnp.float32),
        scratch_types=[
            pltpu.VMEM((4, BLK), jnp.int32),
            pltpu.VMEM((4, BLK, 128), jnp.float32),
            pltpu.VMEM((CH, 128), jnp.float32),
            pltpu.VMEM_SHARED((NPAD, 128), jnp.float32),
            pltpu.SemaphoreType.DMA,
            pltpu.SemaphoreType.DMA,
        ],
    )
    def k(z_hbm, dst_hbm, acc_hbm, dst_r, z_r, zz_v, acc_sh, sem_l, sem_s):
        cid = lax.axis_index("c")
        sid = lax.axis_index("s")

        @pl.loop(0, CH)
        def _(r):
            @pl.loop(0, 128, step=16)
            def _(c0):
                zz_v.at[r, pl.ds(c0, 16)][...] = jnp.zeros((16,), jnp.float32)

        row0 = sid * STRIPE
        for kk in range(NCH):
            pltpu.sync_copy(zz_v, acc_sh.at[pl.ds(row0 + kk * CH, CH), :])
        plsc.subcore_barrier()

        ebase = estart + (cid * NSUB + sid) * epw
        zbase = (cid * NSUB + sid) * epw

        def fire_load(b, r):
            pltpu.async_copy(dst_hbm.at[pl.ds(ebase + b * BLK, BLK)],
                             dst_r.at[r], sem_l)
            pltpu.async_copy(z_hbm.at[pl.ds(zbase + b * BLK, BLK), :],
                             z_r.at[r], sem_l)

        def wait_load(b, r):
            pltpu.make_async_copy(dst_hbm.at[pl.ds(ebase + b * BLK, BLK)],
                                  dst_r.at[r], sem_l).wait()
            pltpu.make_async_copy(
                z_hbm.at[pl.ds(zbase + b * BLK, BLK), :], z_r.at[r],
                sem_l).wait()

        def fire_scat(r):
            pltpu.async_copy(z_r.at[r], acc_sh.at[dst_r.at[r]], sem_s,
                             add=True)

        def wait_scat(r):
            pltpu.make_async_copy(z_r.at[r], acc_sh.at[dst_r.at[r]],
                                  sem_s).wait()

        def step(b, r):
            wait_load(b, r)
            fire_scat(r)
            wait_scat(r)
            fire_load(b + 4, r)

        for b in range(4):
            fire_load(b, b)

        m = (nblk - 5) // 4

        @pl.loop(0, 4 * m, step=4)
        def _(g):
            for r in range(4):
                step(g + r, r)

        for b in range(4 * m, nblk - 5):
            step(b, b % 4)

        r0 = (nblk - 5) % 4
        wait_load(nblk - 5, r0)
        fire_scat(r0)
        wait_scat(r0)
        fire_load(nblk - 1, r0)
        for b in range(nblk - 4, nblk):
            r = b % 4
            wait_load(b, r)
            fire_scat(r)
            wait_scat(r)

        plsc.subcore_barrier()

        out0 = cid * NPAD + row0
        for kk in range(NCH):
            pltpu.sync_copy(acc_sh.at[pl.ds(row0 + kk * CH, CH), :],
                            acc_hbm.at[pl.ds(out0 + kk * CH, CH), :])

    return k(z, dst)


def _fusion(acc1, acc2, cnt1, cnt2, bf2):
    def body(a10, a11, a20, a21, c10, c11, c20, c21, b_ref, o_ref):
        s = (a10[...] + a11[...]) + (a20[...] + a21[...])
        counts = ((c10[...][:, 0:1] + c11[...][:, 0:1])
                  + (c20[...][:, 0:1] + c21[...][:, 0:1]))
        denom = jnp.maximum(counts, 1.0)
        o_ref[...] = jnp.maximum(s / denom + b_ref[...], 0.0)

    bn = 1024
    nb = NPAD // bn
    specs = []
    for _ in range(4):
        specs.append(pl.BlockSpec((bn, 128), lambda i: (i, 0)))
        specs.append(pl.BlockSpec((bn, 128), lambda i: (i + nb, 0)))
    specs.append(pl.BlockSpec((1, UNITS), lambda i: (0, 0)))
    return pl.pallas_call(
        body,
        grid=(nb,),
        in_specs=specs,
        out_specs=pl.BlockSpec((bn, UNITS), lambda i: (i, 0)),
        out_shape=jax.ShapeDtypeStruct((NPAD, UNITS), jnp.float32),
    )(acc1, acc1, acc2, acc2, cnt1, cnt1, cnt2, cnt2, bf2)


def _fusion2(acc1, cnt1, bf2):
    def body(a10, a11, c10, c11, b_ref, o_ref):
        s = a10[...] + a11[...]
        counts = c10[...][:, 0:1] + c11[...][:, 0:1]
        denom = jnp.maximum(counts, 1.0)
        o_ref[...] = jnp.maximum(s / denom + b_ref[...], 0.0)

    bn = 1024
    nb = NPAD // bn
    specs = []
    for _ in range(2):
        specs.append(pl.BlockSpec((bn, 128), lambda i: (i, 0)))
        specs.append(pl.BlockSpec((bn, 128), lambda i: (i + nb, 0)))
    specs.append(pl.BlockSpec((1, UNITS), lambda i: (0, 0)))
    return pl.pallas_call(
        body,
        grid=(nb,),
        in_specs=specs,
        out_specs=pl.BlockSpec((bn, UNITS), lambda i: (i, 0)),
        out_shape=jax.ShapeDtypeStruct((NPAD, UNITS), jnp.float32),
    )(acc1, acc1, cnt1, cnt1, bf2)


MONO = True


@jax.jit
def kernel(node_features, edge_indices, edge_features,
           K0, K1, K2, We1, be1, We2, be2, Wf, bf):
    src = edge_indices[0].astype(jnp.int32)
    dst = edge_indices[1].astype(jnp.int32)

    kcat = jnp.concatenate([K0, K1, K2], axis=1)
    be1r = be1.reshape(1, UNITS)
    be2r = be2.reshape(1, W3)

    if MONO:
        g1, cnt1 = _sc_gather_counts(node_features, src, dst, 0,
                                     NBLK_A + NBLK_B)
        z1 = _edge_stage(edge_features, g1, We1, be1r, We2, be2r, kcat,
                         Wf, 0)
        acc1 = _sc_scatter(z1, dst, 0, NBLK_A + NBLK_B)
        acc2, cnt2 = acc1, cnt1
    else:
        g1, cnt1 = _sc_gather_counts(node_features, src, dst, 0, NBLK_A)
        g2, cnt2 = _sc_gather_counts(node_features, src, dst, HALF_A,
                                     NBLK_B)
        z1 = _edge_stage(edge_features, g1, We1, be1r, We2, be2r, kcat,
                         Wf, 0)
        z2 = _edge_stage(edge_features, g2, We1, be1r, We2, be2r, kcat,
                         Wf, HALF_A)
        acc1 = _sc_scatter(z1, dst, 0, NBLK_A)
        acc2 = _sc_scatter(z2, dst, HALF_A, NBLK_B)

    if MONO:
        out = _fusion2(acc1, cnt1, bf.reshape(1, UNITS))
    else:
        out = _fusion(acc1, acc2, cnt1, cnt2, bf.reshape(1, UNITS))
    return out[:N_NODES]


# speedup vs baseline: 1.0675x; 1.0675x over previous
import functools

import jax
import jax.numpy as jnp
from jax import lax
from jax.experimental import pallas as pl
from jax.experimental.pallas import tpu as pltpu
from jax.experimental.pallas import tpu_sc as plsc

N_NODES = 10000
N_EDGES = 320000
D_FEAT = 128
D_EDGE = 16
UNITS = 128
W3 = 384

NCORE = 2
NSUB = 16
NWORK = NCORE * NSUB
BLK = 80
NBLK_A = 62
NBLK_B = 63
HALF_A = NWORK * BLK * NBLK_A
HALF_B = NWORK * BLK * NBLK_B
NPAD = 10240
STRIPE = NPAD // NSUB
CH = 32
NCH = STRIPE // CH
BE = 2560


def _sc_gather_counts(x, src, dst, estart, nblk):
    mesh = plsc.VectorSubcoreMesh(core_axis_name="c", subcore_axis_name="s")
    epw = BLK * nblk
    nedges = NWORK * epw

    @functools.partial(
        pl.kernel,
        mesh=mesh,
        out_type=[
            jax.ShapeDtypeStruct((nedges, D_FEAT), jnp.float32),
            jax.ShapeDtypeStruct((NCORE * NPAD, 128), jnp.float32),
        ],
        scratch_types=[
            pltpu.VMEM((3, BLK), jnp.int32),
            pltpu.VMEM((3, BLK), jnp.int32),
            pltpu.VMEM((BLK, D_FEAT), jnp.float32),
            pltpu.VMEM((BLK, D_FEAT), jnp.float32),
            pltpu.VMEM((BLK, D_FEAT), jnp.float32),
            pltpu.VMEM((BLK, 128), jnp.float32),
            pltpu.VMEM((CH, 128), jnp.float32),
            pltpu.VMEM_SHARED((NPAD, 128), jnp.float32),
            pltpu.SemaphoreType.DMA,
            pltpu.SemaphoreType.DMA,
            pltpu.SemaphoreType.DMA,
            pltpu.SemaphoreType.DMA,
        ],
    )
    def k(x_hbm, src_hbm, dst_hbm, g_hbm, cnt_hbm,
          src_r, dst_r, g_v0, g_v1, g_v2, ones_v, z_v, cnt_sh,
          sem_g, sem_w, sem_c, sem_i):
        cid = lax.axis_index("c")
        sid = lax.axis_index("s")
        ebase = estart + (cid * NSUB + sid) * epw
        obase = (cid * NSUB + sid) * epw
        gs = (g_v0, g_v1, g_v2)

        @pl.loop(0, CH)
        def _(r):
            @pl.loop(0, 128, step=16)
            def _(c0):
                z_v.at[r, pl.ds(c0, 16)][...] = jnp.zeros((16,), jnp.float32)

        @pl.loop(0, BLK)
        def _(r):
            @pl.loop(0, 128, step=16)
            def _(c0):
                ones_v.at[r, pl.ds(c0, 16)][...] = jnp.ones((16,), jnp.float32)

        row0 = sid * STRIPE
        for kk in range(NCH):
            pltpu.sync_copy(z_v, cnt_sh.at[pl.ds(row0 + kk * CH, CH), :])
        plsc.subcore_barrier()

        def fire_idx(b, r):
            pltpu.async_copy(src_hbm.at[pl.ds(ebase + b * BLK, BLK)],
                             src_r.at[r], sem_i)

        def wait_idx(r):
            pltpu.make_async_copy(src_hbm.at[pl.ds(ebase, BLK)],
                                  src_r.at[r], sem_i).wait()

        def fire_gather(b, r):
            pltpu.async_copy(x_hbm.at[src_r.at[r]], gs[r], sem_g)

        def wait_gather(r):
            pltpu.make_async_copy(x_hbm.at[src_r.at[r]], gs[r], sem_g).wait()

        def fire_write(b, r):
            pltpu.async_copy(gs[r], g_hbm.at[pl.ds(obase + b * BLK, BLK), :],
                             sem_w)

        def wait_write(b, r):
            pltpu.make_async_copy(
                gs[r], g_hbm.at[pl.ds(obase + b * BLK, BLK), :], sem_w).wait()

        def fire_cnt(b, r):
            pltpu.sync_copy(dst_hbm.at[pl.ds(ebase + b * BLK, BLK)],
                            dst_r.at[r])
            pltpu.async_copy(ones_v, cnt_sh.at[dst_r.at[r]], sem_c, add=True)

        def wait_cnt(r):
            pltpu.make_async_copy(ones_v, cnt_sh.at[dst_r.at[r]],
                                  sem_c).wait()

        def step(b, r, prefetch=True):
            r2 = (r + 2) % 3
            wait_gather(r)
            fire_write(b, r)
            if prefetch:
                fire_idx(b + 3, r)
            wait_write(b - 1, r2)
            wait_idx(r2)
            fire_gather(b + 2, r2)
            wait_cnt(r2)
            fire_cnt(b + 2, r2)

        fire_idx(0, 0); fire_idx(1, 1); fire_idx(2, 2)
        wait_idx(0); fire_gather(0, 0); fire_cnt(0, 0)
        wait_idx(1); fire_gather(1, 1); fire_cnt(1, 1)
        wait_gather(0); fire_write(0, 0); fire_idx(3, 0)
        wait_idx(2); fire_gather(2, 2); fire_cnt(2, 2)
        wait_gather(1); fire_write(1, 1); fire_idx(4, 1); wait_write(0, 0)
        wait_idx(0); fire_gather(3, 0); wait_cnt(0); fire_cnt(3, 0)
        wait_gather(2); fire_write(2, 2); fire_idx(5, 2); wait_write(1, 1)
        wait_idx(1); fire_gather(4, 1); wait_cnt(1); fire_cnt(4, 1)

        m = (nblk - 6) // 3

        @pl.loop(3, 3 + 3 * m, step=3)
        def _(g):
            for r in range(3):
                step(g + r, r)

        for b in range(3 + 3 * m, nblk - 3):
            step(b, b % 3)

        step(nblk - 3, (nblk - 3) % 3, prefetch=False)

        ra = (nblk - 2) % 3
        rb = (nblk - 1) % 3
        rc = (nblk - 3) % 3
        wait_gather(ra); fire_write(nblk - 2, ra); wait_write(nblk - 3, rc)
        wait_gather(rb); fire_write(nblk - 1, rb); wait_write(nblk - 2, ra)
        wait_write(nblk - 1, rb)
        wait_cnt(0); wait_cnt(1); wait_cnt(2)

        plsc.subcore_barrier()

        out0 = cid * NPAD + row0
        for kk in range(NCH):
            pltpu.sync_copy(cnt_sh.at[pl.ds(row0 + kk * CH, CH), :],
                            cnt_hbm.at[pl.ds(out0 + kk * CH, CH), :])

    return k(x, src, dst)


def _edge_stage(e_feats, g, we1, be1, we2, be2, kcat, wf, eoff):
    def body(e_ref, g_ref, w1_ref, b1_ref, w2_ref, b2_ref, kc_ref, wf_ref,
             o_ref):
        h = jnp.maximum(
            jnp.dot(e_ref[...], w1_ref[...], preferred_element_type=jnp.float32)
            + b1_ref[...], 0.0)
        w = jnp.dot(h, w2_ref[...], preferred_element_type=jnp.float32) + b2_ref[...]
        gk = jnp.dot(g_ref[...], kc_ref[...], preferred_element_type=jnp.float32)
        o_ref[...] = jnp.dot(w * gk, wf_ref[...],
                             preferred_element_type=jnp.float32)

    nedges = g.shape[0]
    off = eoff // BE
    return pl.pallas_call(
        body,
        grid=(nedges // BE,),
        in_specs=[
            pl.BlockSpec((BE, D_EDGE), lambda i: (i + off, 0)),
            pl.BlockSpec((BE, D_FEAT), lambda i: (i, 0)),
            pl.BlockSpec((D_EDGE, UNITS), lambda i: (0, 0)),
            pl.BlockSpec((1, UNITS), lambda i: (0, 0)),
            pl.BlockSpec((UNITS, W3), lambda i: (0, 0)),
            pl.BlockSpec((1, W3), lambda i: (0, 0)),
            pl.BlockSpec((D_FEAT, W3), lambda i: (0, 0)),
            pl.BlockSpec((W3, UNITS), lambda i: (0, 0)),
        ],
        out_specs=pl.BlockSpec((BE, UNITS), lambda i: (i, 0)),
        out_shape=jax.ShapeDtypeStruct((nedges, UNITS), jnp.float32),
    )(e_feats, g, we1, be1, we2, be2, kcat, wf)


def _sc_scatter(z, dst, estart, nblk):
    mesh = plsc.VectorSubcoreMesh(core_axis_name="c", subcore_axis_name="s")
    epw = BLK * nblk

    @functools.partial(
        pl.kernel,
        mesh=mesh,
        out_type=jax.ShapeDtypeStruct((NCORE * NPAD, 128), jnp.float32),
        scratch_types=[
            pltpu.VMEM((4, BLK), jnp.int32),
            pltpu.VMEM((4, BLK, 128), jnp.float32),
            pltpu.VMEM((CH, 128), jnp.float32),
            pltpu.VMEM_SHARED((NPAD, 128), jnp.float32),
            pltpu.SemaphoreType.DMA,
            pltpu.SemaphoreType.DMA,
        ],
    )
    def k(z_hbm, dst_hbm, acc_hbm, dst_r, z_r, zz_v, acc_sh, sem_l, sem_s):
        cid = lax.axis_index("c")
        sid = lax.axis_index("s")

        @pl.loop(0, CH)
        def _(r):
            @pl.loop(0, 128, step=16)
            def _(c0):
                zz_v.at[r, pl.ds(c0, 16)][...] = jnp.zeros((16,), jnp.float32)

        row0 = sid * STRIPE
        for kk in range(NCH):
            pltpu.sync_copy(zz_v, acc_sh.at[pl.ds(row0 + kk * CH, CH), :])
        plsc.subcore_barrier()

        ebase = estart + (cid * NSUB + sid) * epw
        zbase = (cid * NSUB + sid) * epw

        def fire_load(b, r):
            pltpu.async_copy(dst_hbm.at[pl.ds(ebase + b * BLK, BLK)],
                             dst_r.at[r], sem_l)
            pltpu.async_copy(z_hbm.at[pl.ds(zbase + b * BLK, BLK), :],
                             z_r.at[r], sem_l)

        def wait_load(b, r):
            pltpu.make_async_copy(dst_hbm.at[pl.ds(ebase + b * BLK, BLK)],
                                  dst_r.at[r], sem_l).wait()
            pltpu.make_async_copy(
                z_hbm.at[pl.ds(zbase + b * BLK, BLK), :], z_r.at[r],
                sem_l).wait()

        def fire_scat(r):
            pltpu.async_copy(z_r.at[r], acc_sh.at[dst_r.at[r]], sem_s,
                             add=True)

        def wait_scat(r):
            pltpu.make_async_copy(z_r.at[r], acc_sh.at[dst_r.at[r]],
                                  sem_s).wait()

        def step(b, r):
            wait_load(b, r)
            fire_scat(r)
            wait_scat(r)
            fire_load(b + 4, r)

        for b in range(4):
            fire_load(b, b)

        m = (nblk - 5) // 4

        @pl.loop(0, 4 * m, step=4)
        def _(g):
            for r in range(4):
                step(g + r, r)

        for b in range(4 * m, nblk - 5):
            step(b, b % 4)

        r0 = (nblk - 5) % 4
        wait_load(nblk - 5, r0)
        fire_scat(r0)
        wait_scat(r0)
        fire_load(nblk - 1, r0)
        for b in range(nblk - 4, nblk):
            r = b % 4
            wait_load(b, r)
            fire_scat(r)
            wait_scat(r)

        plsc.subcore_barrier()

        out0 = cid * NPAD + row0
        for kk in range(NCH):
            pltpu.sync_copy(acc_sh.at[pl.ds(row0 + kk * CH, CH), :],
                            acc_hbm.at[pl.ds(out0 + kk * CH, CH), :])

    return k(z, dst)


def _fusion(accs, cnts, bf2):
    n = len(accs)

    def body(*refs):
        a_refs = refs[:2 * n]
        c_refs = refs[2 * n:4 * n]
        b_ref = refs[4 * n]
        o_ref = refs[4 * n + 1]
        s = a_refs[0][...]
        for a in a_refs[1:]:
            s = s + a[...]
        counts = c_refs[0][...][:, 0:1]
        for c in c_refs[1:]:
            counts = counts + c[...][:, 0:1]
        denom = jnp.maximum(counts, 1.0)
        o_ref[...] = jnp.maximum(s / denom + b_ref[...], 0.0)

    bn = 1024
    nb = NPAD // bn
    specs = []
    args = []
    for arr in accs + cnts:
        specs.append(pl.BlockSpec((bn, 128), lambda i: (i, 0)))
        specs.append(pl.BlockSpec((bn, 128), lambda i: (i + nb, 0)))
        args.extend([arr, arr])
    specs.append(pl.BlockSpec((1, UNITS), lambda i: (0, 0)))
    args.append(bf2)
    return pl.pallas_call(
        body,
        grid=(nb,),
        in_specs=specs,
        out_specs=pl.BlockSpec((bn, UNITS), lambda i: (i, 0)),
        out_shape=jax.ShapeDtypeStruct((NPAD, UNITS), jnp.float32),
    )(*args)


PARTS = (41, 42, 42)


@jax.jit
def kernel(node_features, edge_indices, edge_features,
           K0, K1, K2, We1, be1, We2, be2, Wf, bf):
    src = edge_indices[0].astype(jnp.int32)
    dst = edge_indices[1].astype(jnp.int32)

    kcat = jnp.concatenate([K0, K1, K2], axis=1)
    be1r = be1.reshape(1, UNITS)
    be2r = be2.reshape(1, W3)

    starts = []
    e0 = 0
    for nblk in PARTS:
        starts.append(e0)
        e0 += NWORK * BLK * nblk

    gs, cnts = [], []
    for estart, nblk in zip(starts, PARTS):
        g, cnt = _sc_gather_counts(node_features, src, dst, estart, nblk)
        gs.append(g)
        cnts.append(cnt)
    zs = [_edge_stage(edge_features, g, We1, be1r, We2, be2r, kcat, Wf,
                      estart)
          for g, estart in zip(gs, starts)]
    accs = [_sc_scatter(z, dst, estart, nblk)
            for z, estart, nblk in zip(zs, starts, PARTS)]

    out = _fusion(accs, cnts, bf.reshape(1, UNITS))
    return out[:N_NODES]


# speedup vs baseline: 1.0774x; 1.0093x over previous
import functools

import jax
import jax.numpy as jnp
from jax import lax
from jax.experimental import pallas as pl
from jax.experimental.pallas import tpu as pltpu
from jax.experimental.pallas import tpu_sc as plsc

N_NODES = 10000
N_EDGES = 320000
D_FEAT = 128
D_EDGE = 16
UNITS = 128
W3 = 384

NCORE = 2
NSUB = 16
NWORK = NCORE * NSUB
BLK = 80
NBLK_A = 62
NBLK_B = 63
HALF_A = NWORK * BLK * NBLK_A
HALF_B = NWORK * BLK * NBLK_B
NPAD = 10240
STRIPE = NPAD // NSUB
CH = 32
NCH = STRIPE // CH
BE = 2560


def _sc_gather_counts(x, src, dst, estart, nblk):
    mesh = plsc.VectorSubcoreMesh(core_axis_name="c", subcore_axis_name="s")
    epw = BLK * nblk
    nedges = NWORK * epw

    @functools.partial(
        pl.kernel,
        mesh=mesh,
        out_type=[
            jax.ShapeDtypeStruct((nedges, D_FEAT), jnp.float32),
            jax.ShapeDtypeStruct((NCORE * NPAD, 128), jnp.float32),
        ],
        scratch_types=[
            pltpu.VMEM((3, BLK), jnp.int32),
            pltpu.VMEM((3, BLK), jnp.int32),
            pltpu.VMEM((BLK, D_FEAT), jnp.float32),
            pltpu.VMEM((BLK, D_FEAT), jnp.float32),
            pltpu.VMEM((BLK, D_FEAT), jnp.float32),
            pltpu.VMEM((BLK, 128), jnp.float32),
            pltpu.VMEM((CH, 128), jnp.float32),
            pltpu.VMEM_SHARED((NPAD, 128), jnp.float32),
            pltpu.SemaphoreType.DMA,
            pltpu.SemaphoreType.DMA,
            pltpu.SemaphoreType.DMA,
            pltpu.SemaphoreType.DMA,
        ],
    )
    def k(x_hbm, src_hbm, dst_hbm, g_hbm, cnt_hbm,
          src_r, dst_r, g_v0, g_v1, g_v2, ones_v, z_v, cnt_sh,
          sem_g, sem_w, sem_c, sem_i):
        cid = lax.axis_index("c")
        sid = lax.axis_index("s")
        ebase = estart + (cid * NSUB + sid) * epw
        obase = (cid * NSUB + sid) * epw
        gs = (g_v0, g_v1, g_v2)

        @pl.loop(0, CH)
        def _(r):
            @pl.loop(0, 128, step=16)
            def _(c0):
                z_v.at[r, pl.ds(c0, 16)][...] = jnp.zeros((16,), jnp.float32)

        @pl.loop(0, BLK)
        def _(r):
            @pl.loop(0, 128, step=16)
            def _(c0):
                ones_v.at[r, pl.ds(c0, 16)][...] = jnp.ones((16,), jnp.float32)

        row0 = sid * STRIPE
        for kk in range(NCH):
            pltpu.sync_copy(z_v, cnt_sh.at[pl.ds(row0 + kk * CH, CH), :])
        plsc.subcore_barrier()

        def fire_idx(b, r):
            pltpu.async_copy(src_hbm.at[pl.ds(ebase + b * BLK, BLK)],
                             src_r.at[r], sem_i)

        def wait_idx(r):
            pltpu.make_async_copy(src_hbm.at[pl.ds(ebase, BLK)],
                                  src_r.at[r], sem_i).wait()

        def fire_gather(b, r):
            pltpu.async_copy(x_hbm.at[src_r.at[r]], gs[r], sem_g)

        def wait_gather(r):
            pltpu.make_async_copy(x_hbm.at[src_r.at[r]], gs[r], sem_g).wait()

        def fire_write(b, r):
            pltpu.async_copy(gs[r], g_hbm.at[pl.ds(obase + b * BLK, BLK), :],
                             sem_w)

        def wait_write(b, r):
            pltpu.make_async_copy(
                gs[r], g_hbm.at[pl.ds(obase + b * BLK, BLK), :], sem_w).wait()

        def fire_cnt(b, r):
            pltpu.sync_copy(dst_hbm.at[pl.ds(ebase + b * BLK, BLK)],
                            dst_r.at[r])
            pltpu.async_copy(ones_v, cnt_sh.at[dst_r.at[r]], sem_c, add=True)

        def wait_cnt(r):
            pltpu.make_async_copy(ones_v, cnt_sh.at[dst_r.at[r]],
                                  sem_c).wait()

        def step(b, r, prefetch=True):
            r2 = (r + 2) % 3
            wait_gather(r)
            fire_write(b, r)
            if prefetch:
                fire_idx(b + 3, r)
            wait_write(b - 1, r2)
            wait_idx(r2)
            fire_gather(b + 2, r2)
            wait_cnt(r2)
            fire_cnt(b + 2, r2)

        fire_idx(0, 0); fire_idx(1, 1); fire_idx(2, 2)
        wait_idx(0); fire_gather(0, 0); fire_cnt(0, 0)
        wait_idx(1); fire_gather(1, 1); fire_cnt(1, 1)
        wait_gather(0); fire_write(0, 0); fire_idx(3, 0)
        wait_idx(2); fire_gather(2, 2); fire_cnt(2, 2)
        wait_gather(1); fire_write(1, 1); fire_idx(4, 1); wait_write(0, 0)
        wait_idx(0); fire_gather(3, 0); wait_cnt(0); fire_cnt(3, 0)
        wait_gather(2); fire_write(2, 2); fire_idx(5, 2); wait_write(1, 1)
        wait_idx(1); fire_gather(4, 1); wait_cnt(1); fire_cnt(4, 1)

        m = (nblk - 6) // 3

        @pl.loop(3, 3 + 3 * m, step=3)
        def _(g):
            for r in range(3):
                step(g + r, r)

        for b in range(3 + 3 * m, nblk - 3):
            step(b, b % 3)

        step(nblk - 3, (nblk - 3) % 3, prefetch=False)

        ra = (nblk - 2) % 3
        rb = (nblk - 1) % 3
        rc = (nblk - 3) % 3
        wait_gather(ra); fire_write(nblk - 2, ra); wait_write(nblk - 3, rc)
        wait_gather(rb); fire_write(nblk - 1, rb); wait_write(nblk - 2, ra)
        wait_write(nblk - 1, rb)
        wait_cnt(0); wait_cnt(1); wait_cnt(2)

        plsc.subcore_barrier()

        out0 = cid * NPAD + row0
        for kk in range(NCH):
            pltpu.sync_copy(cnt_sh.at[pl.ds(row0 + kk * CH, CH), :],
                            cnt_hbm.at[pl.ds(out0 + kk * CH, CH), :])

    return k(x, src, dst)


def _edge_stage(e_feats, g, we1, be1, we2, be2, kcat, wf, eoff):
    def body(e_ref, g_ref, w1_ref, b1_ref, w2_ref, b2_ref, kc_ref, wf_ref,
             o_ref):
        h = jnp.maximum(
            jnp.dot(e_ref[...], w1_ref[...], preferred_element_type=jnp.float32)
            + b1_ref[...], 0.0)
        w = jnp.dot(h, w2_ref[...], preferred_element_type=jnp.float32) + b2_ref[...]
        gk = jnp.dot(g_ref[...], kc_ref[...], preferred_element_type=jnp.float32)
        o_ref[...] = jnp.dot(w * gk, wf_ref[...],
                             preferred_element_type=jnp.float32)

    nedges = g.shape[0]
    off = eoff // BE
    return pl.pallas_call(
        body,
        grid=(nedges // BE,),
        in_specs=[
            pl.BlockSpec((BE, D_EDGE), lambda i: (i + off, 0)),
            pl.BlockSpec((BE, D_FEAT), lambda i: (i, 0)),
            pl.BlockSpec((D_EDGE, UNITS), lambda i: (0, 0)),
            pl.BlockSpec((1, UNITS), lambda i: (0, 0)),
            pl.BlockSpec((UNITS, W3), lambda i: (0, 0)),
            pl.BlockSpec((1, W3), lambda i: (0, 0)),
            pl.BlockSpec((D_FEAT, W3), lambda i: (0, 0)),
            pl.BlockSpec((W3, UNITS), lambda i: (0, 0)),
        ],
        out_specs=pl.BlockSpec((BE, UNITS), lambda i: (i, 0)),
        out_shape=jax.ShapeDtypeStruct((nedges, UNITS), jnp.float32),
    )(e_feats, g, we1, be1, we2, be2, kcat, wf)


def _sc_scatter(z, dst, estart, nblk):
    mesh = plsc.VectorSubcoreMesh(core_axis_name="c", subcore_axis_name="s")
    epw = BLK * nblk

    @functools.partial(
        pl.kernel,
        mesh=mesh,
        out_type=jax.ShapeDtypeStruct((NCORE * NPAD, 128), jnp.float32),
        scratch_types=[
            pltpu.VMEM((4, BLK), jnp.int32),
            pltpu.VMEM((4, BLK, 128), jnp.float32),
            pltpu.VMEM((CH, 128), jnp.float32),
            pltpu.VMEM_SHARED((NPAD, 128), jnp.float32),
            pltpu.SemaphoreType.DMA,
            pltpu.SemaphoreType.DMA,
        ],
    )
    def k(z_hbm, dst_hbm, acc_hbm, dst_r, z_r, zz_v, acc_sh, sem_l, sem_s):
        cid = lax.axis_index("c")
        sid = lax.axis_index("s")

        @pl.loop(0, CH)
        def _(r):
            @pl.loop(0, 128, step=16)
            def _(c0):
                zz_v.at[r, pl.ds(c0, 16)][...] = jnp.zeros((16,), jnp.float32)

        row0 = sid * STRIPE
        for kk in range(NCH):
            pltpu.sync_copy(zz_v, acc_sh.at[pl.ds(row0 + kk * CH, CH), :])
        plsc.subcore_barrier()

        ebase = estart + (cid * NSUB + sid) * epw
        zbase = (cid * NSUB + sid) * epw

        def fire_load(b, r):
            pltpu.async_copy(dst_hbm.at[pl.ds(ebase + b * BLK, BLK)],
                             dst_r.at[r], sem_l)
            pltpu.async_copy(z_hbm.at[pl.ds(zbase + b * BLK, BLK), :],
                             z_r.at[r], sem_l)

        def wait_load(b, r):
            pltpu.make_async_copy(dst_hbm.at[pl.ds(ebase + b * BLK, BLK)],
                                  dst_r.at[r], sem_l).wait()
            pltpu.make_async_copy(
                z_hbm.at[pl.ds(zbase + b * BLK, BLK), :], z_r.at[r],
                sem_l).wait()

        def fire_scat(r):
            pltpu.async_copy(z_r.at[r], acc_sh.at[dst_r.at[r]], sem_s,
                             add=True)

        def wait_scat(r):
            pltpu.make_async_copy(z_r.at[r], acc_sh.at[dst_r.at[r]],
                                  sem_s).wait()

        def step(b, r, first=False, load=True):
            wait_load(b, r)
            fire_scat(r)
            if not first:
                wait_scat((r + 3) % 4)
            if load:
                fire_load(b + 3, (b + 3) % 4)

        fire_load(0, 0)
        fire_load(1, 1)
        fire_load(2, 2)
        step(0, 0, first=True)
        step(1, 1)
        step(2, 2)
        step(3, 3)

        m = (nblk - 7) // 4

        @pl.loop(4, 4 + 4 * m, step=4)
        def _(g):
            for r in range(4):
                step(g + r, r)

        for b in range(4 + 4 * m, nblk - 3):
            step(b, b % 4)

        for b in range(nblk - 3, nblk):
            step(b, b % 4, load=False)

        wait_scat((nblk - 1) % 4)

        plsc.subcore_barrier()

        out0 = cid * NPAD + row0
        for kk in range(NCH):
            pltpu.sync_copy(acc_sh.at[pl.ds(row0 + kk * CH, CH), :],
                            acc_hbm.at[pl.ds(out0 + kk * CH, CH), :])

    return k(z, dst)


def _fusion(accs, cnts, bf2):
    n = len(accs)

    def body(*refs):
        a_refs = refs[:2 * n]
        c_refs = refs[2 * n:4 * n]
        b_ref = refs[4 * n]
        o_ref = refs[4 * n + 1]
        s = a_refs[0][...]
        for a in a_refs[1:]:
            s = s + a[...]
        counts = c_refs[0][...][:, 0:1]
        for c in c_refs[1:]:
            counts = counts + c[...][:, 0:1]
        denom = jnp.maximum(counts, 1.0)
        o_ref[...] = jnp.maximum(s / denom + b_ref[...], 0.0)

    bn = 1024
    nb = NPAD // bn
    specs = []
    args = []
    for arr in accs + cnts:
        specs.append(pl.BlockSpec((bn, 128), lambda i: (i, 0)))
        specs.append(pl.BlockSpec((bn, 128), lambda i: (i + nb, 0)))
        args.extend([arr, arr])
    specs.append(pl.BlockSpec((1, UNITS), lambda i: (0, 0)))
    args.append(bf2)
    return pl.pallas_call(
        body,
        grid=(nb,),
        in_specs=specs,
        out_specs=pl.BlockSpec((bn, UNITS), lambda i: (i, 0)),
        out_shape=jax.ShapeDtypeStruct((NPAD, UNITS), jnp.float32),
    )(*args)


PARTS = (62, 63)


@jax.jit
def kernel(node_features, edge_indices, edge_features,
           K0, K1, K2, We1, be1, We2, be2, Wf, bf):
    src = edge_indices[0].astype(jnp.int32)
    dst = edge_indices[1].astype(jnp.int32)

    kcat = jnp.concatenate([K0, K1, K2], axis=1)
    be1r = be1.reshape(1, UNITS)
    be2r = be2.reshape(1, W3)

    starts = []
    e0 = 0
    for nblk in PARTS:
        starts.append(e0)
        e0 += NWORK * BLK * nblk

    gs, cnts = [], []
    for estart, nblk in zip(starts, PARTS):
        g, cnt = _sc_gather_counts(node_features, src, dst, estart, nblk)
        gs.append(g)
        cnts.append(cnt)
    zs = [_edge_stage(edge_features, g, We1, be1r, We2, be2r, kcat, Wf,
                      estart)
          for g, estart in zip(gs, starts)]
    accs = [_sc_scatter(z, dst, estart, nblk)
            for z, estart, nblk in zip(zs, starts, PARTS)]

    out = _fusion(accs, cnts, bf.reshape(1, UNITS))
    return out[:N_NODES]
